# baseline (speedup 1.0000x reference)
import functools

import jax
import jax.numpy as jnp
from jax import lax
from jax.experimental import pallas as pl
from jax.experimental.pallas import tpu as pltpu
from jax.experimental.pallas import tpu_sc as plsc

NE = 64
D = 768
DFF = 2048
T = 2048
ALIGN = 8
TPAD = T + NE * ALIGN
TILE = 32

_NC = 2
_NS = 16
_NW = _NC * _NS
CHUNK = T // _NW


def _route_body(x_ref, rw_ref, p_ref, w_ref, offs_ref, cnt_ref):
    x = x_ref[...]
    logits = jnp.dot(x, rw_ref[...], preferred_element_type=jnp.float32)
    m = jnp.max(logits, axis=1, keepdims=True)
    ex = jnp.exp(logits - m)
    s = jnp.sum(ex, axis=1, keepdims=True)
    w_ref[...] = jnp.broadcast_to(1.0 / s, (T, 128))

    lane = lax.broadcasted_iota(jnp.int32, (T, NE), 1)
    eid = jnp.min(jnp.where(logits == m, lane, NE), axis=1, keepdims=True)
    oh = (lane == eid).astype(jnp.float32)

    r = lax.broadcasted_iota(jnp.int32, (T, T), 0)
    c = lax.broadcasted_iota(jnp.int32, (T, T), 1)
    ltri = (c < r).astype(jnp.float32)
    rank = jnp.dot(ltri, oh, preferred_element_type=jnp.float32)

    counts = jnp.sum(oh, axis=0, keepdims=True)
    cpad = jnp.floor((counts + (ALIGN - 1)) / ALIGN) * ALIGN
    rr = lax.broadcasted_iota(jnp.int32, (NE, NE + 1), 0)
    cc = lax.broadcasted_iota(jnp.int32, (NE, NE + 1), 1)
    utri = (rr < cc).astype(jnp.float32)
    offs = jnp.dot(cpad, utri, preferred_element_type=jnp.float32)

    pos = jnp.sum(oh * (offs[:, :NE] + rank), axis=1, keepdims=True)
    p_ref[...] = pos.astype(jnp.int32)
    offs_ref[...] = offs.astype(jnp.int32)
    cnt_ref[...] = counts.astype(jnp.int32)


_route = pl.pallas_call(
    _route_body,
    out_shape=[
        jax.ShapeDtypeStruct((T, 1), jnp.int32),
        jax.ShapeDtypeStruct((T, 128), jnp.float32),
        jax.ShapeDtypeStruct((1, NE + 1), jnp.int32),
        jax.ShapeDtypeStruct((1, NE), jnp.int32),
    ],
)


@functools.cache
def _sc_kernels():
    mesh = plsc.VectorSubcoreMesh(
        core_axis_name="c", subcore_axis_name="s", num_cores=_NC, num_subcores=_NS
    )

    @functools.partial(
        pl.kernel,
        mesh=mesh,
        out_type=[
            jax.ShapeDtypeStruct((TPAD, D), jnp.float32),
            jax.ShapeDtypeStruct((TPAD, 128), jnp.float32),
        ],
        scratch_types=[
            pltpu.VMEM((CHUNK,), jnp.int32),
            pltpu.VMEM((CHUNK, D), jnp.float32),
            pltpu.VMEM((CHUNK, 128), jnp.float32),
            pltpu.SemaphoreType.DMA,
            pltpu.SemaphoreType.DMA,
        ],
    )
    def dispatch(x_hbm, p_hbm, w_hbm, xs_hbm, ws_hbm, idx_v, rows_v, wv, sem1, sem2):
        wid = lax.axis_index("s") * _NC + lax.axis_index("c")
        base = wid * CHUNK
        pltpu.sync_copy(p_hbm.at[pl.ds(base, CHUNK)], idx_v)
        pltpu.sync_copy(x_hbm.at[pl.ds(base, CHUNK)], rows_v)
        pltpu.sync_copy(w_hbm.at[pl.ds(base, CHUNK)], wv)
        cp1 = pltpu.async_copy(rows_v, xs_hbm.at[idx_v], sem1)
        cp2 = pltpu.async_copy(wv, ws_hbm.at[idx_v], sem2)
        cp1.wait()
        cp2.wait()

    @functools.partial(
        pl.kernel,
        mesh=mesh,
        out_type=jax.ShapeDtypeStruct((T, D), jnp.float32),
        scratch_types=[
            pltpu.VMEM((CHUNK,), jnp.int32),
            pltpu.VMEM((CHUNK, D), jnp.float32),
            pltpu.SemaphoreType.DMA,
        ],
    )
    def combine(y_hbm, p_hbm, out_hbm, idx_v, rows_v, sem):
        wid = lax.axis_index("s") * _NC + lax.axis_index("c")
        base = wid * CHUNK
        pltpu.sync_copy(p_hbm.at[pl.ds(base, CHUNK)], idx_v)
        pltpu.async_copy(y_hbm.at[idx_v], rows_v, sem).wait()
        pltpu.sync_copy(rows_v, out_hbm.at[pl.ds(base, CHUNK)])

    return dispatch, combine


NF = 4
DFFC = DFF // NF
NBUF = 7
NCHUNK = NE * NF


def _gemm_body(offs_ref, cnt_ref, xs_ref, ws_ref, gate_hbm, up_hbm, down_hbm,
               y_ref, gbuf, ubuf, dbuf, sems):
    def copies(k, slot):
        e = lax.div(k, NF)
        f = lax.rem(k, NF)
        cg = pltpu.make_async_copy(
            gate_hbm.at[e, :, pl.ds(f * DFFC, DFFC)], gbuf.at[slot],
            sems.at[slot])
        cu = pltpu.make_async_copy(
            up_hbm.at[e, :, pl.ds(f * DFFC, DFFC)], ubuf.at[slot],
            sems.at[slot])
        cd = pltpu.make_async_copy(
            down_hbm.at[e, pl.ds(f * DFFC, DFFC), :], dbuf.at[slot],
            sems.at[slot])
        return cg, cu, cd

    def issue(k):
        cg, cu, cd = copies(k, lax.rem(k, NBUF))
        cg.start()
        cu.start()
        cd.start()

    for k in range(NBUF - 1):
        issue(k)

    def step(k, carry):
        @pl.when(k + NBUF - 1 < NCHUNK)
        def _():
            issue(k + NBUF - 1)

        slot = lax.rem(k, NBUF)
        cg, cu, cd = copies(k, slot)
        cg.wait()
        cu.wait()
        cd.wait()

        e = lax.div(k, NF)
        f = lax.rem(k, NF)
        start = offs_ref[0, e]
        n = cnt_ref[0, e]
        gw = gbuf[slot]
        uw = ubuf[slot]
        dw = dbuf[slot]

        def body(i, c):
            r0 = pl.multiple_of(start + i * TILE, ALIGN)
            xt = xs_ref[pl.ds(r0, TILE), :]
            g = jnp.dot(xt, gw, preferred_element_type=jnp.float32)
            u = jnp.dot(xt, uw, preferred_element_type=jnp.float32)
            h = g * (1.0 / (1.0 + jnp.exp(-g))) * u
            y = jnp.dot(h, dw, preferred_element_type=jnp.float32)
            rows = pl.ds(r0, TILE)

            @pl.when(f == 0)
            def _():
                y_ref[rows, :] = y

            @pl.when(f == NF - 1)
            def _():
                wst = ws_ref[rows, 0:1]
                acc = y if NF == 1 else y_ref[rows, :] + y
                y_ref[rows, :] = acc * wst

            @pl.when(jnp.logical_and(f > 0, f < NF - 1))
            def _():
                y_ref[rows, :] = y_ref[rows, :] + y

            return c

        lax.fori_loop(0, (n + TILE - 1) // TILE, body, 0)
        return carry

    lax.fori_loop(0, NCHUNK, step, 0)


_grouped = pl.pallas_call(
    _gemm_body,
    in_specs=[
        pl.BlockSpec(memory_space=pltpu.SMEM),
        pl.BlockSpec(memory_space=pltpu.SMEM),
        pl.BlockSpec(memory_space=pltpu.VMEM),
        pl.BlockSpec(memory_space=pltpu.VMEM),
        pl.BlockSpec(memory_space=pl.ANY),
        pl.BlockSpec(memory_space=pl.ANY),
        pl.BlockSpec(memory_space=pl.ANY),
    ],
    out_specs=pl.BlockSpec(memory_space=pltpu.VMEM),
    out_shape=jax.ShapeDtypeStruct((TPAD, D), jnp.float32),
    scratch_shapes=[
        pltpu.VMEM((NBUF, D, DFFC), jnp.float32),
        pltpu.VMEM((NBUF, D, DFFC), jnp.float32),
        pltpu.VMEM((NBUF, DFFC, D), jnp.float32),
        pltpu.SemaphoreType.DMA((NBUF,)),
    ],
)


def kernel(x, router_w, gate_w, up_w, down_w):
    B_, L_, D_ = x.shape
    dispatch, combine = _sc_kernels()
    xf = x.reshape(T, D)
    p, w, offs, cnt = _route(xf, router_w)
    pf = p.reshape(T)
    xs, ws = dispatch(xf, pf, w)
    y = _grouped(offs, cnt, xs, ws, gate_w, up_w, down_w)
    out = combine(y, pf)
    return out.reshape(B_, L_, D_)


# speedup vs baseline: 1.0084x; 1.0084x over previous
import functools

import jax
import jax.numpy as jnp
from jax import lax
from jax.experimental import pallas as pl
from jax.experimental.pallas import tpu as pltpu
from jax.experimental.pallas import tpu_sc as plsc

NE = 64
D = 768
DFF = 2048
T = 2048
ALIGN = 8
TPAD = T + NE * ALIGN
TILE = 32

_NC = 2
_NS = 16
_NW = _NC * _NS
CHUNK = T // _NW


def _route_body(x_ref, rw_ref, p_ref, w_ref, offs_ref, cnt_ref):
    x = x_ref[...]
    logits = jnp.dot(x, rw_ref[...], preferred_element_type=jnp.float32)
    m = jnp.max(logits, axis=1, keepdims=True)
    ex = jnp.exp(logits - m)
    s = jnp.sum(ex, axis=1, keepdims=True)
    w_ref[...] = jnp.broadcast_to(1.0 / s, (T, 128))

    lane = lax.broadcasted_iota(jnp.int32, (T, NE), 1)
    eid = jnp.min(jnp.where(logits == m, lane, NE), axis=1, keepdims=True)
    oh = (lane == eid).astype(jnp.float32)

    r = lax.broadcasted_iota(jnp.int32, (T, T), 0)
    c = lax.broadcasted_iota(jnp.int32, (T, T), 1)
    ltri = (c < r).astype(jnp.float32)
    rank = jnp.dot(ltri, oh, preferred_element_type=jnp.float32)

    counts = jnp.sum(oh, axis=0, keepdims=True)
    cpad = jnp.floor((counts + (ALIGN - 1)) / ALIGN) * ALIGN
    rr = lax.broadcasted_iota(jnp.int32, (NE, NE + 1), 0)
    cc = lax.broadcasted_iota(jnp.int32, (NE, NE + 1), 1)
    utri = (rr < cc).astype(jnp.float32)
    offs = jnp.dot(cpad, utri, preferred_element_type=jnp.float32)

    pos = jnp.sum(oh * (offs[:, :NE] + rank), axis=1, keepdims=True)
    p_ref[...] = pos.astype(jnp.int32)
    offs_ref[...] = offs.astype(jnp.int32)
    cnt_ref[...] = counts.astype(jnp.int32)


_route = pl.pallas_call(
    _route_body,
    out_shape=[
        jax.ShapeDtypeStruct((T, 1), jnp.int32),
        jax.ShapeDtypeStruct((T, 128), jnp.float32),
        jax.ShapeDtypeStruct((1, NE + 1), jnp.int32),
        jax.ShapeDtypeStruct((1, NE), jnp.int32),
    ],
)


@functools.cache
def _sc_kernels():
    mesh = plsc.VectorSubcoreMesh(
        core_axis_name="c", subcore_axis_name="s", num_cores=_NC, num_subcores=_NS
    )

    @functools.partial(
        pl.kernel,
        mesh=mesh,
        out_type=[
            jax.ShapeDtypeStruct((TPAD, D), jnp.float32),
            jax.ShapeDtypeStruct((TPAD, 128), jnp.float32),
        ],
        scratch_types=[
            pltpu.VMEM((CHUNK,), jnp.int32),
            pltpu.VMEM((CHUNK, D), jnp.float32),
            pltpu.VMEM((CHUNK, 128), jnp.float32),
            pltpu.SemaphoreType.DMA,
            pltpu.SemaphoreType.DMA,
        ],
    )
    def dispatch(x_hbm, p_hbm, w_hbm, xs_hbm, ws_hbm, idx_v, rows_v, wv, sem1, sem2):
        wid = lax.axis_index("s") * _NC + lax.axis_index("c")
        base = wid * CHUNK
        pltpu.sync_copy(p_hbm.at[pl.ds(base, CHUNK)], idx_v)
        pltpu.sync_copy(x_hbm.at[pl.ds(base, CHUNK)], rows_v)
        pltpu.sync_copy(w_hbm.at[pl.ds(base, CHUNK)], wv)
        cp1 = pltpu.async_copy(rows_v, xs_hbm.at[idx_v], sem1)
        cp2 = pltpu.async_copy(wv, ws_hbm.at[idx_v], sem2)
        cp1.wait()
        cp2.wait()

    @functools.partial(
        pl.kernel,
        mesh=mesh,
        out_type=jax.ShapeDtypeStruct((T, D), jnp.float32),
        scratch_types=[
            pltpu.VMEM((CHUNK,), jnp.int32),
            pltpu.VMEM((CHUNK, D), jnp.float32),
            pltpu.SemaphoreType.DMA,
        ],
    )
    def combine(y_hbm, p_hbm, out_hbm, idx_v, rows_v, sem):
        wid = lax.axis_index("s") * _NC + lax.axis_index("c")
        base = wid * CHUNK
        pltpu.sync_copy(p_hbm.at[pl.ds(base, CHUNK)], idx_v)
        pltpu.async_copy(y_hbm.at[idx_v], rows_v, sem).wait()
        pltpu.sync_copy(rows_v, out_hbm.at[pl.ds(base, CHUNK)])

    return dispatch, combine


NF = 2
DFFC = DFF // NF
NBUF = 3
NCHUNK = NE * NF


def _gemm_body(offs_ref, cnt_ref, xs_ref, ws_ref, gate_hbm, up_hbm, down_hbm,
               y_ref, gbuf, ubuf, dbuf, sems):
    def copies(k, slot):
        e = lax.div(k, NF)
        f = lax.rem(k, NF)
        cg = pltpu.make_async_copy(
            gate_hbm.at[e, :, pl.ds(f * DFFC, DFFC)], gbuf.at[slot],
            sems.at[slot])
        cu = pltpu.make_async_copy(
            up_hbm.at[e, :, pl.ds(f * DFFC, DFFC)], ubuf.at[slot],
            sems.at[slot])
        cd = pltpu.make_async_copy(
            down_hbm.at[e, pl.ds(f * DFFC, DFFC), :], dbuf.at[slot],
            sems.at[slot])
        return cg, cu, cd

    def issue(k):
        cg, cu, cd = copies(k, lax.rem(k, NBUF))
        cg.start()
        cu.start()
        cd.start()

    for k in range(NBUF - 1):
        issue(k)

    def step(k, carry):
        @pl.when(k + NBUF - 1 < NCHUNK)
        def _():
            issue(k + NBUF - 1)

        slot = lax.rem(k, NBUF)
        cg, cu, cd = copies(k, slot)
        cg.wait()
        cu.wait()
        cd.wait()

        e = lax.div(k, NF)
        f = lax.rem(k, NF)
        start = offs_ref[0, e]
        n = cnt_ref[0, e]
        gw = gbuf[slot]
        uw = ubuf[slot]
        dw = dbuf[slot]

        def body(i, c):
            r0 = pl.multiple_of(start + i * TILE, ALIGN)
            xt = xs_ref[pl.ds(r0, TILE), :]
            g = jnp.dot(xt, gw, preferred_element_type=jnp.float32)
            u = jnp.dot(xt, uw, preferred_element_type=jnp.float32)
            h = g * (1.0 / (1.0 + jnp.exp(-g))) * u
            y = jnp.dot(h, dw, preferred_element_type=jnp.float32)
            rows = pl.ds(r0, TILE)

            @pl.when(f == 0)
            def _():
                y_ref[rows, :] = y

            @pl.when(f == NF - 1)
            def _():
                wst = ws_ref[rows, 0:1]
                acc = y if NF == 1 else y_ref[rows, :] + y
                y_ref[rows, :] = acc * wst

            @pl.when(jnp.logical_and(f > 0, f < NF - 1))
            def _():
                y_ref[rows, :] = y_ref[rows, :] + y

            return c

        lax.fori_loop(0, (n + TILE - 1) // TILE, body, 0)
        return carry

    lax.fori_loop(0, NCHUNK, step, 0)


_grouped = pl.pallas_call(
    _gemm_body,
    in_specs=[
        pl.BlockSpec(memory_space=pltpu.SMEM),
        pl.BlockSpec(memory_space=pltpu.SMEM),
        pl.BlockSpec(memory_space=pltpu.VMEM),
        pl.BlockSpec(memory_space=pltpu.VMEM),
        pl.BlockSpec(memory_space=pl.ANY),
        pl.BlockSpec(memory_space=pl.ANY),
        pl.BlockSpec(memory_space=pl.ANY),
    ],
    out_specs=pl.BlockSpec(memory_space=pltpu.VMEM),
    out_shape=jax.ShapeDtypeStruct((TPAD, D), jnp.float32),
    scratch_shapes=[
        pltpu.VMEM((NBUF, D, DFFC), jnp.float32),
        pltpu.VMEM((NBUF, D, DFFC), jnp.float32),
        pltpu.VMEM((NBUF, DFFC, D), jnp.float32),
        pltpu.SemaphoreType.DMA((NBUF,)),
    ],
)


def kernel(x, router_w, gate_w, up_w, down_w):
    B_, L_, D_ = x.shape
    dispatch, combine = _sc_kernels()
    xf = x.reshape(T, D)
    p, w, offs, cnt = _route(xf, router_w)
    pf = p.reshape(T)
    xs, ws = dispatch(xf, pf, w)
    y = _grouped(offs, cnt, xs, ws, gate_w, up_w, down_w)
    out = combine(y, pf)
    return out.reshape(B_, L_, D_)


# speedup vs baseline: 1.0113x; 1.0029x over previous
import functools

import jax
import jax.numpy as jnp
from jax import lax
from jax.experimental import pallas as pl
from jax.experimental.pallas import tpu as pltpu
from jax.experimental.pallas import tpu_sc as plsc

NE = 64
D = 768
DFF = 2048
T = 2048
ALIGN = 8
TPAD = T + NE * ALIGN
TILE = 32
NB = 16
BL = T // NB

_NC = 2
_NS = 16
_NW = _NC * _NS
CHUNK = T // _NW


def _route_body(x_ref, rw_ref, p_ref, w_ref, offs_ref, cnt_ref):
    x = x_ref[...]
    logits = jnp.dot(x, rw_ref[...], preferred_element_type=jnp.float32)
    m = jnp.max(logits, axis=1, keepdims=True)
    ex = jnp.exp(logits - m)
    s = jnp.sum(ex, axis=1, keepdims=True)
    w_ref[...] = jnp.broadcast_to(1.0 / s, (T, 128))

    lane = lax.broadcasted_iota(jnp.int32, (T, NE), 1)
    eid = jnp.min(jnp.where(logits == m, lane, NE), axis=1, keepdims=True)
    oh = (lane == eid).astype(jnp.float32)

    tri = (
        lax.broadcasted_iota(jnp.int32, (BL, BL), 1)
        < lax.broadcasted_iota(jnp.int32, (BL, BL), 0)
    ).astype(jnp.float32)
    ranks = []
    bsums = []
    for b in range(NB):
        ohb = oh[b * BL:(b + 1) * BL, :]
        ranks.append(jnp.dot(tri, ohb, preferred_element_type=jnp.float32))
        bsums.append(jnp.sum(ohb, axis=0, keepdims=True))
    bs = jnp.concatenate(bsums, axis=0)
    tri_nb = (
        lax.broadcasted_iota(jnp.int32, (NB, NB), 1)
        < lax.broadcasted_iota(jnp.int32, (NB, NB), 0)
    ).astype(jnp.float32)
    pref = jnp.dot(tri_nb, bs, preferred_element_type=jnp.float32)
    rank = jnp.concatenate(
        [ranks[b] + pref[b:b + 1, :] for b in range(NB)], axis=0
    )

    counts = jnp.sum(bs, axis=0, keepdims=True)
    cpad = jnp.floor((counts + (ALIGN - 1)) / ALIGN) * ALIGN
    rr = lax.broadcasted_iota(jnp.int32, (NE, NE + 1), 0)
    cc = lax.broadcasted_iota(jnp.int32, (NE, NE + 1), 1)
    utri = (rr < cc).astype(jnp.float32)
    offs = jnp.dot(cpad, utri, preferred_element_type=jnp.float32)

    pos = jnp.sum(oh * (offs[:, :NE] + rank), axis=1, keepdims=True)
    p_ref[...] = pos.astype(jnp.int32)
    offs_ref[...] = offs.astype(jnp.int32)
    cnt_ref[...] = counts.astype(jnp.int32)


_route = pl.pallas_call(
    _route_body,
    out_shape=[
        jax.ShapeDtypeStruct((T, 1), jnp.int32),
        jax.ShapeDtypeStruct((T, 128), jnp.float32),
        jax.ShapeDtypeStruct((1, NE + 1), jnp.int32),
        jax.ShapeDtypeStruct((1, NE), jnp.int32),
    ],
)


@functools.cache
def _sc_kernels():
    mesh = plsc.VectorSubcoreMesh(
        core_axis_name="c", subcore_axis_name="s", num_cores=_NC, num_subcores=_NS
    )

    @functools.partial(
        pl.kernel,
        mesh=mesh,
        out_type=[
            jax.ShapeDtypeStruct((TPAD, D), jnp.float32),
            jax.ShapeDtypeStruct((TPAD, 128), jnp.float32),
        ],
        scratch_types=[
            pltpu.VMEM((CHUNK,), jnp.int32),
            pltpu.VMEM((CHUNK, D), jnp.float32),
            pltpu.VMEM((CHUNK, 128), jnp.float32),
            pltpu.SemaphoreType.DMA,
            pltpu.SemaphoreType.DMA,
        ],
    )
    def dispatch(x_hbm, p_hbm, w_hbm, xs_hbm, ws_hbm, idx_v, rows_v, wv, sem1, sem2):
        wid = lax.axis_index("s") * _NC + lax.axis_index("c")
        base = wid * CHUNK
        pltpu.sync_copy(p_hbm.at[pl.ds(base, CHUNK)], idx_v)
        pltpu.sync_copy(x_hbm.at[pl.ds(base, CHUNK)], rows_v)
        pltpu.sync_copy(w_hbm.at[pl.ds(base, CHUNK)], wv)
        cp1 = pltpu.async_copy(rows_v, xs_hbm.at[idx_v], sem1)
        cp2 = pltpu.async_copy(wv, ws_hbm.at[idx_v], sem2)
        cp1.wait()
        cp2.wait()

    @functools.partial(
        pl.kernel,
        mesh=mesh,
        out_type=jax.ShapeDtypeStruct((T, D), jnp.float32),
        scratch_types=[
            pltpu.VMEM((CHUNK,), jnp.int32),
            pltpu.VMEM((CHUNK, D), jnp.float32),
            pltpu.SemaphoreType.DMA,
        ],
    )
    def combine(y_hbm, p_hbm, out_hbm, idx_v, rows_v, sem):
        wid = lax.axis_index("s") * _NC + lax.axis_index("c")
        base = wid * CHUNK
        pltpu.sync_copy(p_hbm.at[pl.ds(base, CHUNK)], idx_v)
        pltpu.async_copy(y_hbm.at[idx_v], rows_v, sem).wait()
        pltpu.sync_copy(rows_v, out_hbm.at[pl.ds(base, CHUNK)])

    return dispatch, combine


NF = 2
DFFC = DFF // NF
NBUF = 3
NCHUNK = NE * NF


def _gemm_body(offs_ref, cnt_ref, xs_ref, ws_ref, gate_hbm, up_hbm, down_hbm,
               y_ref, gbuf, ubuf, dbuf, sems):
    def copies(k, slot):
        e = lax.div(k, NF)
        f = lax.rem(k, NF)
        cg = pltpu.make_async_copy(
            gate_hbm.at[e, :, pl.ds(f * DFFC, DFFC)], gbuf.at[slot],
            sems.at[slot])
        cu = pltpu.make_async_copy(
            up_hbm.at[e, :, pl.ds(f * DFFC, DFFC)], ubuf.at[slot],
            sems.at[slot])
        cd = pltpu.make_async_copy(
            down_hbm.at[e, pl.ds(f * DFFC, DFFC), :], dbuf.at[slot],
            sems.at[slot])
        return cg, cu, cd

    def issue(k):
        cg, cu, cd = copies(k, lax.rem(k, NBUF))
        cg.start()
        cu.start()
        cd.start()

    for k in range(NBUF - 1):
        issue(k)

    def step(k, carry):
        @pl.when(k + NBUF - 1 < NCHUNK)
        def _():
            issue(k + NBUF - 1)

        slot = lax.rem(k, NBUF)
        cg, cu, cd = copies(k, slot)
        cg.wait()
        cu.wait()
        cd.wait()

        e = lax.div(k, NF)
        f = lax.rem(k, NF)
        start = offs_ref[0, e]
        n = cnt_ref[0, e]
        gw = gbuf[slot]
        uw = ubuf[slot]
        dw = dbuf[slot]

        def body(i, c):
            r0 = pl.multiple_of(start + i * TILE, ALIGN)
            xt = xs_ref[pl.ds(r0, TILE), :]
            g = jnp.dot(xt, gw, preferred_element_type=jnp.float32)
            u = jnp.dot(xt, uw, preferred_element_type=jnp.float32)
            h = g * (1.0 / (1.0 + jnp.exp(-g))) * u
            y = jnp.dot(h, dw, preferred_element_type=jnp.float32)
            rows = pl.ds(r0, TILE)

            @pl.when(f == 0)
            def _():
                y_ref[rows, :] = y

            @pl.when(f == NF - 1)
            def _():
                wst = ws_ref[rows, 0:1]
                acc = y if NF == 1 else y_ref[rows, :] + y
                y_ref[rows, :] = acc * wst

            @pl.when(jnp.logical_and(f > 0, f < NF - 1))
            def _():
                y_ref[rows, :] = y_ref[rows, :] + y

            return c

        lax.fori_loop(0, (n + TILE - 1) // TILE, body, 0)
        return carry

    lax.fori_loop(0, NCHUNK, step, 0)


_grouped = pl.pallas_call(
    _gemm_body,
    in_specs=[
        pl.BlockSpec(memory_space=pltpu.SMEM),
        pl.BlockSpec(memory_space=pltpu.SMEM),
        pl.BlockSpec(memory_space=pltpu.VMEM),
        pl.BlockSpec(memory_space=pltpu.VMEM),
        pl.BlockSpec(memory_space=pl.ANY),
        pl.BlockSpec(memory_space=pl.ANY),
        pl.BlockSpec(memory_space=pl.ANY),
    ],
    out_specs=pl.BlockSpec(memory_space=pltpu.VMEM),
    out_shape=jax.ShapeDtypeStruct((TPAD, D), jnp.float32),
    scratch_shapes=[
        pltpu.VMEM((NBUF, D, DFFC), jnp.float32),
        pltpu.VMEM((NBUF, D, DFFC), jnp.float32),
        pltpu.VMEM((NBUF, DFFC, D), jnp.float32),
        pltpu.SemaphoreType.DMA((NBUF,)),
    ],
)


def kernel(x, router_w, gate_w, up_w, down_w):
    B_, L_, D_ = x.shape
    dispatch, combine = _sc_kernels()
    xf = x.reshape(T, D)
    p, w, offs, cnt = _route(xf, router_w)
    pf = p.reshape(T)
    xs, ws = dispatch(xf, pf, w)
    y = _grouped(offs, cnt, xs, ws, gate_w, up_w, down_w)
    out = combine(y, pf)
    return out.reshape(B_, L_, D_)
